# baseline (speedup 1.0000x reference)
import functools

import jax
import jax.numpy as jnp
from jax.experimental import pallas as pl

_EPS = 1e-5


def _router_block(x_ref, emb_ref, ones_ref, out_ref, *, temp):
    x = x_ref[...]
    emb = emb_ref[...]
    wones = ones_ref[...]

    inv_d = 1.0 / x.shape[1]

    a = jax.lax.dot_general(
        wones, x, (((1,), (1,)), ((), ())), preferred_element_type=jnp.float32,
        precision=jax.lax.Precision.HIGHEST,
    )
    q = jax.lax.dot_general(
        wones, x * x, (((1,), (1,)), ((), ())),
        preferred_element_type=jnp.float32,
        precision=jax.lax.Precision.HIGHEST,
    )

    m = a[0:1, :] * inv_d
    var = q[0:1, :] * inv_d - m * m
    rstd = jax.lax.rsqrt(var + _EPS)
    mrstd = m * rstd

    rows = jnp.concatenate([mrstd, rstd, mrstd, rstd, mrstd, rstd, mrstd, rstd],
                           axis=0)
    cols = rows.T
    mrstd_col = cols[:, 0:1]
    rstd_col = cols[:, 1:2]

    xn = x * rstd_col - mrstd_col

    sim = jax.lax.dot_general(
        emb, xn, (((1,), (1,)), ((), ())), preferred_element_type=jnp.float32
    )

    iota = jax.lax.broadcasted_iota(jnp.int32, sim.shape, 0)
    max1 = jnp.max(sim, axis=0, keepdims=True)
    idx1 = jnp.min(jnp.where(sim == max1, iota, 8), axis=0, keepdims=True)
    masked = jnp.where(iota == idx1, -jnp.inf, sim)
    max2 = jnp.max(masked, axis=0, keepdims=True)
    idx2 = jnp.min(jnp.where(masked == max2, iota, 8), axis=0, keepdims=True)

    e2 = jnp.exp((max2 - max1) / temp)
    denom = 1.0 + e2
    p1 = 1.0 / denom
    p2 = e2 / denom

    i1f = idx1.astype(jnp.float32)
    i2f = idx2.astype(jnp.float32)
    out_ref[...] = jnp.concatenate([i1f, i2f, p1, p2, i1f, i2f, p1, p2], axis=0)


def kernel(input, expert_embeddings):
    b, s, d = input.shape
    e = expert_embeddings.shape[0]
    n = b * s
    x = input.reshape(n, d)

    wones = jnp.zeros((8, d), jnp.float32).at[0, :].set(1.0)

    block_t = 2048
    grid = (n // block_t,)
    temp = float(d) ** 0.5

    out = pl.pallas_call(
        functools.partial(_router_block, temp=temp),
        grid=grid,
        in_specs=[
            pl.BlockSpec((block_t, d), lambda i: (i, 0)),
            pl.BlockSpec((e, d), lambda i: (0, 0)),
            pl.BlockSpec((8, d), lambda i: (0, 0)),
        ],
        out_specs=pl.BlockSpec((8, block_t), lambda i: (0, i)),
        out_shape=jax.ShapeDtypeStruct((8, n), jnp.float32),
    )(x, expert_embeddings, wones)

    idx = out[0:2, :].astype(jnp.int32).T.reshape(b, s, 2)
    probs = out[2:4, :].T.reshape(b, s, 2)
    return idx, probs


# speedup vs baseline: 4.6047x; 4.6047x over previous
import functools

import jax
import jax.numpy as jnp
from jax.experimental import pallas as pl

_EPS = 1e-5


def _router_block(x_ref, emb_ref, out_ref, *, temp):
    x = x_ref[...]
    emb = emb_ref[...]

    m = jnp.mean(x, axis=1, keepdims=True)
    c = x - m
    v = jnp.mean(c * c, axis=1, keepdims=True)
    xn = c * jax.lax.rsqrt(v + _EPS)

    sim = jax.lax.dot_general(
        emb, xn, (((1,), (1,)), ((), ())), preferred_element_type=jnp.float32
    )

    iota = jax.lax.broadcasted_iota(jnp.int32, sim.shape, 0)
    max1 = jnp.max(sim, axis=0, keepdims=True)
    idx1 = jnp.min(jnp.where(sim == max1, iota, 8), axis=0, keepdims=True)
    masked = jnp.where(iota == idx1, -jnp.inf, sim)
    max2 = jnp.max(masked, axis=0, keepdims=True)
    idx2 = jnp.min(jnp.where(masked == max2, iota, 8), axis=0, keepdims=True)

    e2 = jnp.exp((max2 - max1) / temp)
    denom = 1.0 + e2
    p1 = 1.0 / denom
    p2 = e2 / denom

    i1f = idx1.astype(jnp.float32)
    i2f = idx2.astype(jnp.float32)
    out_ref[...] = jnp.concatenate([i1f, i2f, p1, p2, i1f, i2f, p1, p2], axis=0)


def kernel(input, expert_embeddings):
    b, s, d = input.shape
    e = expert_embeddings.shape[0]
    n = b * s
    x = input.reshape(n, d)

    block_t = 2048
    grid = (n // block_t,)
    temp = float(d) ** 0.5

    out = pl.pallas_call(
        functools.partial(_router_block, temp=temp),
        grid=grid,
        in_specs=[
            pl.BlockSpec((block_t, d), lambda i: (i, 0)),
            pl.BlockSpec((e, d), lambda i: (0, 0)),
        ],
        out_specs=pl.BlockSpec((8, block_t), lambda i: (0, i)),
        out_shape=jax.ShapeDtypeStruct((8, n), jnp.float32),
    )(x, expert_embeddings)

    idx = out[0:2, :].astype(jnp.int32).T.reshape(b, s, 2)
    probs = out[2:4, :].T.reshape(b, s, 2)
    return idx, probs


# speedup vs baseline: 4.7853x; 1.0392x over previous
import functools

import jax
import jax.numpy as jnp
from jax.experimental import pallas as pl

_EPS = 1e-5


def _router_block(x_ref, emb_ref, out_ref, *, temp):
    x = x_ref[...]
    emb = emb_ref[...]

    m = jnp.mean(x, axis=1, keepdims=True)
    q = jnp.mean(x * x, axis=1, keepdims=True)
    v = q - m * m
    xn = ((x - m) * jax.lax.rsqrt(v + _EPS)).astype(jnp.bfloat16)

    sim = jax.lax.dot_general(
        emb.astype(jnp.bfloat16), xn, (((1,), (1,)), ((), ())),
        preferred_element_type=jnp.float32,
    )

    iota = jax.lax.broadcasted_iota(jnp.int32, sim.shape, 0)
    max1 = jnp.max(sim, axis=0, keepdims=True)
    idx1 = jnp.min(jnp.where(sim == max1, iota, 8), axis=0, keepdims=True)
    masked = jnp.where(iota == idx1, -jnp.inf, sim)
    max2 = jnp.max(masked, axis=0, keepdims=True)
    idx2 = jnp.min(jnp.where(masked == max2, iota, 8), axis=0, keepdims=True)

    e2 = jnp.exp((max2 - max1) / temp)
    denom = 1.0 + e2
    p1 = 1.0 / denom
    p2 = e2 / denom

    i1f = idx1.astype(jnp.float32)
    i2f = idx2.astype(jnp.float32)
    out_ref[...] = jnp.concatenate([i1f, i2f, p1, p2, i1f, i2f, p1, p2], axis=0)


def kernel(input, expert_embeddings):
    b, s, d = input.shape
    e = expert_embeddings.shape[0]
    n = b * s
    x = input.reshape(n, d)

    block_t = 2048
    grid = (n // block_t,)
    temp = float(d) ** 0.5

    out = pl.pallas_call(
        functools.partial(_router_block, temp=temp),
        grid=grid,
        in_specs=[
            pl.BlockSpec((block_t, d), lambda i: (i, 0)),
            pl.BlockSpec((e, d), lambda i: (0, 0)),
        ],
        out_specs=pl.BlockSpec((8, block_t), lambda i: (0, i)),
        out_shape=jax.ShapeDtypeStruct((8, n), jnp.float32),
    )(x, expert_embeddings)

    idx = out[0:2, :].astype(jnp.int32).T.reshape(b, s, 2)
    probs = out[2:4, :].T.reshape(b, s, 2)
    return idx, probs
